# baseline (speedup 1.0000x reference)
import functools

import jax
import jax.numpy as jnp
from jax import lax
from jax.experimental import pallas as pl
from jax.experimental.pallas import tpu as pltpu
from jax.experimental.pallas import tpu_sc as plsc

_NUM_CLASSES = 100000
_EMBED_DIM = 64
_MULT = 26
_BATCH = 4096

_NC = 2
_NS = 16
_NW = _NC * _NS
_LANES = 16

_B = _BATCH * _MULT
_B_PER_W = _B // _NW
_CHUNK = 128
_N_CHUNKS = _B_PER_W // _CHUNK


def _emb_kernel(idx_hbm, table_hbm, out_hbm, idx_v, rows_v, sem):
  wid = lax.axis_index("s") * _NC + lax.axis_index("c")
  base = wid * _B_PER_W

  pltpu.sync_copy(idx_hbm.at[pl.ds(base, _B_PER_W)], idx_v)

  lanes = lax.iota(jnp.int32, _LANES)

  def add_offsets(i, carry):
    s = pl.multiple_of(i * _LANES, _LANES)
    col = lax.rem(lanes + i * _LANES, _MULT)
    idx_v[pl.ds(s, _LANES)] = idx_v[pl.ds(s, _LANES)] + col * _NUM_CLASSES
    return carry

  lax.fori_loop(0, _B_PER_W // _LANES, add_offsets, 0)

  def do_chunk(ci, carry):
    off = pl.multiple_of(ci * _CHUNK, _CHUNK)
    pltpu.async_copy(
        table_hbm.at[idx_v.at[pl.ds(off, _CHUNK)]], rows_v, sem
    ).wait()
    pltpu.sync_copy(rows_v, out_hbm.at[pl.ds(base + off, _CHUNK)])
    return carry

  lax.fori_loop(0, _N_CHUNKS, do_chunk, 0)


def kernel(to_emb, table):
  idx_flat = to_emb.reshape(_B).astype(jnp.int32)

  mesh = plsc.VectorSubcoreMesh(core_axis_name="c", subcore_axis_name="s")
  run = functools.partial(
      pl.kernel,
      mesh=mesh,
      out_type=jax.ShapeDtypeStruct((_B, _EMBED_DIM), jnp.float32),
      scratch_types=[
          pltpu.VMEM((_B_PER_W,), jnp.int32),
          pltpu.VMEM((_CHUNK, _EMBED_DIM), jnp.float32),
          pltpu.SemaphoreType.DMA,
      ],
      compiler_params=pltpu.CompilerParams(use_tc_tiling_on_sc=False),
  )(_emb_kernel)

  out = run(idx_flat, table)
  return out.reshape(_BATCH, _MULT, _EMBED_DIM)


# speedup vs baseline: 1.0128x; 1.0128x over previous
import functools

import jax
import jax.numpy as jnp
from jax import lax
from jax.experimental import pallas as pl
from jax.experimental.pallas import tpu as pltpu
from jax.experimental.pallas import tpu_sc as plsc

_NUM_CLASSES = 100000
_EMBED_DIM = 64
_MULT = 26
_BATCH = 4096

_NC = 2
_NS = 16
_NW = _NC * _NS
_LANES = 16

_B = _BATCH * _MULT
_B_PER_W = _B // _NW
_CHUNK = 104
_N_CHUNKS = _B_PER_W // _CHUNK
_NBUF = 8


def _emb_kernel(idx_hbm, table_hbm, out_hbm, idx_v, rows_v, *sems):
  wid = lax.axis_index("s") * _NC + lax.axis_index("c")
  base = wid * _B_PER_W

  pltpu.sync_copy(idx_hbm.at[pl.ds(base, _B_PER_W)], idx_v)

  lanes = lax.iota(jnp.int32, _LANES)

  def add_offsets(i, carry):
    s = pl.multiple_of(i * 4 * _LANES, _LANES)
    for u in range(4):
      col = lax.rem(lanes + (i * 4 + u) * _LANES, _MULT)
      sl = pl.ds(s + u * _LANES, _LANES)
      idx_v[sl] = idx_v[sl] + col * _NUM_CLASSES
    return carry

  lax.fori_loop(0, _B_PER_W // (4 * _LANES), add_offsets, 0)

  def fire_gather(ci, b):
    off = pl.multiple_of(ci * _CHUNK, 8)
    pltpu.async_copy(
        table_hbm.at[idx_v.at[pl.ds(off, _CHUNK)]], rows_v.at[b], sems[b]
    )

  def retire(ci, b):
    off = pl.multiple_of(ci * _CHUNK, 8)
    pltpu.make_async_copy(
        table_hbm.at[idx_v.at[pl.ds(off, _CHUNK)]], rows_v.at[b], sems[b]
    ).wait()
    pltpu.sync_copy(rows_v.at[b], out_hbm.at[pl.ds(base + off, _CHUNK)])

  for b in range(_NBUF):
    fire_gather(b, b)

  def group(g, carry):
    for b in range(_NBUF):
      ci = g * _NBUF + b
      retire(ci, b)
      fire_gather(ci + _NBUF, b)
    return carry

  lax.fori_loop(0, _N_CHUNKS // _NBUF - 1, group, 0)

  for b in range(_NBUF):
    retire(_N_CHUNKS - _NBUF + b, b)


def kernel(to_emb, table):
  idx_flat = to_emb.reshape(_B).astype(jnp.int32)

  mesh = plsc.VectorSubcoreMesh(core_axis_name="c", subcore_axis_name="s")
  run = functools.partial(
      pl.kernel,
      mesh=mesh,
      out_type=jax.ShapeDtypeStruct((_B, _EMBED_DIM), jnp.float32),
      scratch_types=[
          pltpu.VMEM((_B_PER_W,), jnp.int32),
          pltpu.VMEM((_NBUF, _CHUNK, _EMBED_DIM), jnp.float32),
      ] + [pltpu.SemaphoreType.DMA] * _NBUF,
      compiler_params=pltpu.CompilerParams(use_tc_tiling_on_sc=False),
  )(_emb_kernel)

  out = run(idx_flat, table)
  return out.reshape(_BATCH, _MULT, _EMBED_DIM)


# speedup vs baseline: 1.8248x; 1.8017x over previous
import functools

import jax
import jax.numpy as jnp
from jax import lax
from jax.experimental import pallas as pl
from jax.experimental.pallas import tpu as pltpu
from jax.experimental.pallas import tpu_sc as plsc

_NUM_CLASSES = 100000
_EMBED_DIM = 64
_MULT = 26
_BATCH = 4096

_NC = 2
_NS = 16
_NW = _NC * _NS
_L = 16

_B = _BATCH * _MULT
_V = _MULT * _NUM_CLASSES
_RANGE = 81920
_BLK = 512
_NBLK = _RANGE // _BLK
_TAIL = _V - (_V % _L and 0) - 64
_CLAMP = _V - _BLK - 64
_SLICE = _B // _NS
_BKTCAP = 320
_EXCAP = 512
_LISTCAP = 6144


def _next_col(col):
  return col + jnp.where(col >= _MULT - _L, 16 - _MULT, 16)


def _rank16(keys, lanes):
  ks, perm = plsc.sort_key_val(keys, lanes)
  prev = ks.at[jnp.maximum(lanes - 1, 0)].get(mode="promise_in_bounds")
  flag = (ks != prev) | (lanes == 0)
  segstart = plsc.cummax(jnp.where(flag, lanes, 0))
  return ks, perm, lanes - segstart


def _scan_one(v, pos0, col, myc, tail, bcnt_v, bq_v, bp_v, lanes):
  ones = jnp.ones((_L,), jnp.int32)
  q = v + col * _NUM_CLASSES
  own = lax.shift_right_logical(
      lax.shift_right_logical(q, 12) * 6554, 17)
  m = (lax.shift_right_logical(own, 4) == myc) & (q < tail)
  d = own & 15
  ks, perm, rank = _rank16(jnp.where(m, d, 31), lanes)
  qs = q.at[perm].get(mode="promise_in_bounds")
  ms = m.astype(jnp.int32).at[perm].get(mode="promise_in_bounds") > 0
  ofs = plsc.load_gather(bcnt_v, [ks])
  dest = ks * _BKTCAP + ofs + rank
  plsc.store_scatter(bq_v, [dest], qs, mask=ms)
  plsc.store_scatter(bp_v, [dest], pos0 + perm, mask=ms)
  plsc.addupdate_scatter(bcnt_v, [ks], ones, mask=ms)
  return _next_col(col)


def _emb_kernel(tt_hbm, idx_hbm, out_hbm, inbuf, bq_v, bp_v, bcnt_v,
                pairs_q, pairs_p, cnt256_v, sq_v, sp_v,
                hist_v, bstart_v, bcur_v, blockbuf, outstage,
                exq_sh, exp_sh, excnt_sh,
                sem_in, sem_a, sem_b, sem_out):
  myc = lax.axis_index("c")
  mys = lax.axis_index("s")
  w = myc * _NS + mys
  base = w * _RANGE
  upper = jnp.minimum(base + _RANGE, _TAIL)
  lanes = lax.iota(jnp.int32, _L)
  zeros = jnp.zeros((_L,), jnp.int32)

  blk_sems = (sem_a, sem_b)

  def fire_block(b, buf):
    s_b = jnp.minimum(base + b * _BLK, _CLAMP)
    s_b = pl.multiple_of(s_b, 8)
    for r in range(8):
      pltpu.async_copy(
          tt_hbm.at[pl.ds(8 * r, 8), pl.ds(s_b, _BLK)],
          blockbuf.at[buf, pl.ds(8 * r, 8), :],
          blk_sems[buf],
      )

  def wait_block(b, buf):
    s_b = jnp.minimum(base + b * _BLK, _CLAMP)
    s_b = pl.multiple_of(s_b, 8)
    for r in range(8):
      pltpu.make_async_copy(
          tt_hbm.at[pl.ds(8 * r, 8), pl.ds(s_b, _BLK)],
          blockbuf.at[buf, pl.ds(8 * r, 8), :],
          blk_sems[buf],
      ).wait()

  fire_block(0, 0)
  fire_block(1, 1)

  slice_base = mys * _SLICE
  pltpu.async_copy(idx_hbm.at[pl.ds(slice_base, _SLICE)], inbuf, sem_in)

  for t in range(11):
    hist_v[pl.ds(16 * t, 16)] = zeros
  bcnt_v[pl.ds(0, 16)] = zeros
  bcnt_v[pl.ds(16, 16)] = zeros

  pltpu.make_async_copy(idx_hbm.at[pl.ds(slice_base, _SLICE)],
                        inbuf, sem_in).wait()

  def scan_body(j, col):
    sv = pl.multiple_of(j * _L, _L)
    v = inbuf[pl.ds(sv, _L)]
    pos0 = slice_base + j * _L
    return _scan_one(v, pos0, col, myc, _TAIL, bcnt_v, bq_v, bp_v, lanes)

  lax.fori_loop(0, _SLICE // _L, scan_body, lanes)

  exbase = pl.multiple_of(mys * (_NS * _EXCAP), 8)
  for d in range(_NS):
    pltpu.async_copy(bq_v.at[pl.ds(d * _BKTCAP, _BKTCAP)],
                     exq_sh.at[pl.ds(exbase + d * _EXCAP, _BKTCAP)], sem_in)
    pltpu.async_copy(bp_v.at[pl.ds(d * _BKTCAP, _BKTCAP)],
                     exp_sh.at[pl.ds(exbase + d * _EXCAP, _BKTCAP)], sem_in)
  pltpu.sync_copy(bcnt_v.at[pl.ds(0, _NS)],
                  excnt_sh.at[pl.ds(mys * _NS, _NS)])
  for d in range(_NS):
    pltpu.make_async_copy(bq_v.at[pl.ds(d * _BKTCAP, _BKTCAP)],
                          exq_sh.at[pl.ds(exbase + d * _EXCAP, _BKTCAP)],
                          sem_in).wait()
    pltpu.make_async_copy(bp_v.at[pl.ds(d * _BKTCAP, _BKTCAP)],
                          exp_sh.at[pl.ds(exbase + d * _EXCAP, _BKTCAP)],
                          sem_in).wait()
  plsc.subcore_barrier()
  pltpu.sync_copy(excnt_sh, cnt256_v)

  def fire_seg(src, buf):
    soff = pl.multiple_of((src * _NS) * _EXCAP + mys * _EXCAP, 8)
    pltpu.async_copy(exq_sh.at[pl.ds(soff, _EXCAP)], pairs_q.at[buf], sem_in)
    pltpu.async_copy(exp_sh.at[pl.ds(soff, _EXCAP)], pairs_p.at[buf], sem_in)

  def wait_seg(src, buf):
    soff = pl.multiple_of((src * _NS) * _EXCAP + mys * _EXCAP, 8)
    pltpu.make_async_copy(exq_sh.at[pl.ds(soff, _EXCAP)], pairs_q.at[buf],
                          sem_in).wait()
    pltpu.make_async_copy(exp_sh.at[pl.ds(soff, _EXCAP)], pairs_p.at[buf],
                          sem_in).wait()

  fire_seg(0, 0)
  for src in range(_NS):
    buf = src % 2
    wait_seg(src, buf)
    if src + 1 < _NS:
      fire_seg(src + 1, 1 - buf)
    scnt = plsc.load_gather(
        cnt256_v, [jnp.full((_L,), 1, jnp.int32) * (src * _NS + mys)])

    def merge_body(i, carry, _buf=buf, _scnt=scnt):
      so = pl.multiple_of(i * _L, _L)
      qv = pairs_q[_buf, pl.ds(so, _L)]
      m = (i * _L + lanes) < _scnt
      blk = lax.shift_right_logical(qv - base, 9)
      plsc.addupdate_scatter(hist_v, [jnp.where(m, blk, 175)],
                             jnp.ones((_L,), jnp.int32), mask=m)
      return carry

    lax.fori_loop(0, _EXCAP // _L, merge_body, 0)

  carry = jnp.int32(0)
  for t in range(11):
    h = hist_v[pl.ds(16 * t, 16)]
    hp = (h + 15) & ~15
    inc = plsc.cumsum(hp)
    start = carry + inc - hp
    bstart_v[pl.ds(16 * t, 16)] = start
    bcur_v[pl.ds(16 * t, 16)] = start
    carry = carry + lax.reduce_max(inc, axes=(0,))

  fire_seg(0, 0)
  for src in range(_NS):
    buf = src % 2
    wait_seg(src, buf)
    if src + 1 < _NS:
      fire_seg(src + 1, 1 - buf)
    scnt = plsc.load_gather(
        cnt256_v, [jnp.full((_L,), 1, jnp.int32) * (src * _NS + mys)])

    def regroup(i, carry, _buf=buf, _scnt=scnt):
      so = pl.multiple_of(i * _L, _L)
      qv = pairs_q[_buf, pl.ds(so, _L)]
      pv = pairs_p[_buf, pl.ds(so, _L)]
      valid = (i * _L + lanes) < _scnt
      blk = lax.shift_right_logical(qv - base, 9)
      ks, perm, rank = _rank16(jnp.where(valid, blk, 175), lanes)
      qs = qv.at[perm].get(mode="promise_in_bounds")
      ps = pv.at[perm].get(mode="promise_in_bounds")
      vs = valid.astype(jnp.int32).at[perm].get(mode="promise_in_bounds") > 0
      ofs = plsc.load_gather(bcur_v, [ks])
      dest = ofs + rank
      plsc.store_scatter(sq_v, [dest], qs, mask=vs)
      plsc.store_scatter(sp_v, [dest], ps, mask=vs)
      plsc.addupdate_scatter(bcur_v, [ks], jnp.ones((_L,), jnp.int32),
                             mask=vs)
      return carry

    lax.fori_loop(0, _EXCAP // _L, regroup, 0)

  def process_block(b, buf):
    wait_block(b, buf)
    bsp = jnp.full((_L,), b, jnp.int32)
    n_b = lax.reduce_max(plsc.load_gather(hist_v, [bsp]), axes=(0,))
    st_b = pl.multiple_of(
        lax.reduce_max(plsc.load_gather(bstart_v, [bsp]), axes=(0,)), _L)
    s_b = jnp.minimum(base + b * _BLK, _CLAMP)

    def extract(t, carry):
      ofs = pl.multiple_of(st_b + t * _L, _L)
      qv = sq_v[pl.ds(ofs, _L)]
      pv = sp_v[pl.ds(ofs, _L)]
      vmask = lanes < (n_b - t * _L)
      local = lax.clamp(jnp.int32(0), qv - s_b, jnp.int32(_BLK - 1))
      slot = lax.rem(t, 8)
      for ent in range(_L):
        le = local.at[jnp.full((_L,), ent, jnp.int32)].get(
            mode="promise_in_bounds")
        for j in range(4):
          rows = j * 16 + lanes
          outstage[slot, ent, pl.ds(16 * j, 16)] = plsc.load_gather(
              blockbuf.at[buf], [rows, le])
      pdest = jnp.where(vmask, pv, _B + lanes)
      pltpu.async_copy(outstage.at[slot], out_hbm.at[pdest], sem_out)
      return carry

    trips = lax.shift_right_logical(n_b + 15, 4)
    lax.fori_loop(0, trips, extract, 0)

    def drain(t, carry):
      ofs = pl.multiple_of(st_b + t * _L, _L)
      pv = sp_v[pl.ds(ofs, _L)]
      vmask = lanes < (n_b - t * _L)
      pdest = jnp.where(vmask, pv, _B + lanes)
      slot = lax.rem(t, 8)
      pltpu.make_async_copy(outstage.at[slot], out_hbm.at[pdest],
                            sem_out).wait()
      return carry

    lax.fori_loop(0, trips, drain, 0)

  def group(g, carry):
    for sub in range(2):
      b = g * 2 + sub
      process_block(b, sub)

      @pl.when(b + 2 < _NBLK)
      def _():
        fire_block(b + 2, sub)

    return carry

  lax.fori_loop(0, _NBLK // 2, group, 0)


def kernel(to_emb, table):
  tt = table.T
  idx_flat = to_emb.reshape(_B).astype(jnp.int32)

  mesh = plsc.VectorSubcoreMesh(core_axis_name="c", subcore_axis_name="s")
  run = functools.partial(
      pl.kernel,
      mesh=mesh,
      out_type=jax.ShapeDtypeStruct((_B + 16, 2 * _EMBED_DIM), jnp.float32),
      scratch_types=[
          pltpu.VMEM((_SLICE,), jnp.int32),
          pltpu.VMEM((_NS * _BKTCAP,), jnp.int32),
          pltpu.VMEM((_NS * _BKTCAP,), jnp.int32),
          pltpu.VMEM((32,), jnp.int32),
          pltpu.VMEM((2, _EXCAP), jnp.int32),
          pltpu.VMEM((2, _EXCAP), jnp.int32),
          pltpu.VMEM((256,), jnp.int32),
          pltpu.VMEM((_LISTCAP,), jnp.int32),
          pltpu.VMEM((_LISTCAP,), jnp.int32),
          pltpu.VMEM((176,), jnp.int32),
          pltpu.VMEM((176,), jnp.int32),
          pltpu.VMEM((176,), jnp.int32),
          pltpu.VMEM((2, _EMBED_DIM, _BLK), jnp.float32),
          pltpu.VMEM((8, _L, 2 * _EMBED_DIM), jnp.float32),
          pltpu.VMEM_SHARED((_NS * _NS * _EXCAP,), jnp.int32),
          pltpu.VMEM_SHARED((_NS * _NS * _EXCAP,), jnp.int32),
          pltpu.VMEM_SHARED((256,), jnp.int32),
          pltpu.SemaphoreType.DMA,
          pltpu.SemaphoreType.DMA,
          pltpu.SemaphoreType.DMA,
          pltpu.SemaphoreType.DMA,
      ],
      compiler_params=pltpu.CompilerParams(
          use_tc_tiling_on_sc=True, needs_layout_passes=False),
  )(_emb_kernel)

  out_sc = run(tt, idx_flat)[: _B, : _EMBED_DIM].reshape(
      _BATCH, _MULT, _EMBED_DIM)

  raw25 = to_emb[:, 25]
  m = raw25 >= (_NUM_CLASSES - 64)
  tail_tbl = lax.slice(table, (_TAIL, 0), (_V, _EMBED_DIM))
  oh = (raw25[:, None] == ((_NUM_CLASSES - 64) +
                           jnp.arange(64, dtype=to_emb.dtype))[None, :])
  fix = oh.astype(jnp.float32) @ tail_tbl
  colmask = (jnp.arange(_MULT) == (_MULT - 1))[None, :, None]
  return jnp.where(colmask & m[:, None, None], fix[:, None, :], out_sc)
